# Initial kernel scaffold; baseline (speedup 1.0000x reference)
import jax
import jax.numpy as jnp
from jax.experimental import pallas as pl


def kernel(x, edge_index, edge_attr, U_W, U_b, M_W, M_b):
    raise NotImplementedError("write your pallas kernel here")


# speedup vs baseline: 4.3959x; 4.3959x over previous
import functools

import jax
import jax.numpy as jnp
from jax import lax
from jax.experimental import pallas as pl
from jax.experimental.pallas import tpu as pltpu
from jax.experimental.pallas import tpu_sc as plsc

_NC = 1
_NS = 16
_NW = _NC * _NS
_CHUNK = 128
_SB = 40


def _sc_segment_sum(n_pad, d, nch, gather):
    rows_per_tile = n_pad // _NS
    mesh = plsc.VectorSubcoreMesh(core_axis_name="c", subcore_axis_name="s",
                                  num_cores=_NC)

    def body(*refs):
        if gather:
            (x_hbm, src_hbm, dst_hbm, z_hbm, out_hbm,
             sidx, didx, vals, accum, gsem) = refs
        else:
            (val_hbm, dst_hbm, z_hbm, out_hbm,
             didx, vals, accum, gsem) = refs
        c = lax.axis_index("c")
        s = lax.axis_index("s")
        wid = s * _NC + c
        r0 = s * rows_per_tile

        pltpu.sync_copy(z_hbm, vals)

        def zrow(i, carry):
            pltpu.sync_copy(vals, accum.at[pl.ds(r0 + i * _CHUNK, _CHUNK)])
            return carry
        lax.fori_loop(0, rows_per_tile // _CHUNK, zrow, 0)
        plsc.subcore_barrier()

        def superchunk(i, carry):
            pltpu.sync_copy(dst_hbm.at[wid].at[pl.ds(i * _SB, _SB)], didx)
            if gather:
                pltpu.sync_copy(src_hbm.at[wid].at[pl.ds(i * _SB, _SB)],
                                sidx)

            def chunk(j, carry2):
                if gather:
                    pltpu.async_copy(x_hbm.at[sidx.at[j]], vals, gsem).wait()
                else:
                    pltpu.async_copy(val_hbm.at[wid, i * _SB + j], vals,
                                     gsem).wait()
                pltpu.sync_copy(vals, accum.at[didx.at[j]], add=True)
                return carry2
            lax.fori_loop(0, _SB, chunk, 0)
            return carry
        lax.fori_loop(0, nch // _SB, superchunk, 0)
        plsc.subcore_barrier()

        pltpu.sync_copy(accum.at[pl.ds(r0, rows_per_tile)],
                        out_hbm.at[c].at[pl.ds(r0, rows_per_tile)])

    scratch = [
        pltpu.VMEM((_SB, _CHUNK), jnp.int32),
        pltpu.VMEM((_CHUNK, d), jnp.float32),
        pltpu.VMEM_SHARED((n_pad, d), jnp.float32),
        pltpu.SemaphoreType.DMA,
    ]
    if gather:
        scratch = [pltpu.VMEM((_SB, _CHUNK), jnp.int32)] + scratch
    return pl.kernel(
        body,
        out_type=jax.ShapeDtypeStruct((_NC, n_pad, d), jnp.float32),
        mesh=mesh,
        scratch_types=scratch,
    )


def _update_body(x_ref, g_ref, sd_ref, u1_ref, u2_ref, w3_ref,
                 m1_ref, m2_ref, mb_ref, o_ref):
    xb = x_ref[...]
    g = g_ref[0]
    sd = sd_ref[0]
    deg = sd[:, 16:17]
    f32 = jnp.float32
    msg = (deg * jnp.dot(xb, u1_ref[...], preferred_element_type=f32)
           + jnp.dot(g, u2_ref[...], preferred_element_type=f32)
           + jnp.dot(sd, w3_ref[...], preferred_element_type=f32))
    o = (jnp.dot(xb, m1_ref[...], preferred_element_type=f32)
         + jnp.dot(msg, m2_ref[...], preferred_element_type=f32)
         + mb_ref[...])
    o_ref[...] = jnp.maximum(o, 0.0)


def _tc_update(xp, gparts, sdparts, u1, u2, w3, m1, m2, mb):
    n_pad, d = xp.shape
    blk = 1024
    grid = n_pad // blk
    full = lambda i: (0, 0)
    return pl.pallas_call(
        _update_body,
        grid=(grid,),
        in_specs=[
            pl.BlockSpec((blk, d), lambda i: (i, 0)),
            pl.BlockSpec((_NC, blk, d), lambda i: (0, i, 0)),
            pl.BlockSpec((_NC, blk, d), lambda i: (0, i, 0)),
            pl.BlockSpec((d, d), full),
            pl.BlockSpec((d, d), full),
            pl.BlockSpec((d, d), full),
            pl.BlockSpec((d, d), full),
            pl.BlockSpec((d, d), full),
            pl.BlockSpec((1, d), full),
        ],
        out_specs=pl.BlockSpec((blk, d), lambda i: (i, 0)),
        out_shape=jax.ShapeDtypeStruct((n_pad, d), jnp.float32),
    )(xp, gparts, sdparts, u1, u2, w3, m1, m2, mb)


def kernel(x, edge_index, edge_attr, U_W, U_b, M_W, M_b):
    n, d = x.shape
    e = edge_index.shape[1]
    de = edge_attr.shape[1]
    t = U_W.shape[0]

    sbe = _CHUNK * _SB
    ept = -(-e // (_NW * sbe)) * sbe
    nch = ept // _CHUNK
    ep = ept * _NW
    pad = ep - e
    n_pad = -(-(n + 16) // (_CHUNK * _NS)) * (_CHUNK * _NS)
    junk = n_pad - n

    src = edge_index[0]
    dst = edge_index[1]
    ar = jnp.arange(pad, dtype=jnp.int32)
    src_p = jnp.concatenate([src, ar % n]).reshape(_NW, nch, _CHUNK)
    dst_p = jnp.concatenate([dst, n + ar % junk]).reshape(_NW, nch, _CHUNK)

    aug = jnp.concatenate(
        [edge_attr, jnp.ones((e, 1), jnp.float32),
         jnp.zeros((e, d - de - 1), jnp.float32)], axis=1)
    aug_p = jnp.pad(aug, ((0, pad), (0, 0))).reshape(_NW, nch, _CHUNK, d)

    xp = jnp.pad(x, ((0, junk), (0, 0)))
    zeros = jnp.zeros((_CHUNK, d), jnp.float32)

    seg_gather = _sc_segment_sum(n_pad, d, nch, gather=True)
    seg_linear = _sc_segment_sum(n_pad, d, nch, gather=False)

    sdparts = seg_linear(aug_p, dst_p, zeros)

    for k in range(t):
        u1 = U_W[k, :d]
        u2 = U_W[k, d:2 * d]
        w3 = jnp.zeros((d, d), jnp.float32)
        w3 = w3.at[:de].set(U_W[k, 2 * d:]).at[de].set(U_b[k])
        m1 = M_W[k, :d]
        m2 = M_W[k, d:]
        mb = M_b[k][None, :]
        gparts = seg_gather(xp, src_p, dst_p, zeros)
        xp = _tc_update(xp, gparts, sdparts, u1, u2, w3, m1, m2, mb)

    return xp[:n]


# speedup vs baseline: 6.0167x; 1.3687x over previous
import jax
import jax.numpy as jnp
from jax import lax
from jax.experimental import pallas as pl
from jax.experimental.pallas import tpu as pltpu
from jax.experimental.pallas import tpu_sc as plsc

_NC = 1
_NS = 16
_CHUNK = 128
_SB = 40


def _sc_gather_segsum(n_pad, dh, nch):
    rows_per_tile = n_pad // _NS
    mesh = plsc.VectorSubcoreMesh(core_axis_name="c", subcore_axis_name="s",
                                  num_cores=_NC)

    def body(x_hbm, src_hbm, dst_hbm, z_hbm, out_hbm,
             sidx, didx, vals0, vals1, accum, gsem0, gsem1):
        c = lax.axis_index("c")
        s = lax.axis_index("s")
        r0 = s * rows_per_tile

        pltpu.sync_copy(z_hbm, vals0)

        def zrow(i, carry):
            pltpu.sync_copy(vals0, accum.at[pl.ds(r0 + i * _CHUNK, _CHUNK)])
            return carry
        lax.fori_loop(0, rows_per_tile // _CHUNK, zrow, 0)
        plsc.subcore_barrier()

        def superchunk(i, carry):
            pltpu.sync_copy(dst_hbm.at[s].at[pl.ds(i * _SB, _SB)], didx)
            pltpu.sync_copy(src_hbm.at[s].at[pl.ds(i * _SB, _SB)], sidx)
            pltpu.async_copy(x_hbm.at[c].at[sidx.at[0]], vals0, gsem0)

            def pair(p, carry2):
                j0 = 2 * p
                pltpu.async_copy(x_hbm.at[c].at[sidx.at[j0 + 1]], vals1,
                                 gsem1)
                pltpu.make_async_copy(x_hbm.at[c].at[sidx.at[j0]], vals0,
                                      gsem0).wait()
                pltpu.sync_copy(vals0, accum.at[didx.at[j0]], add=True)

                @pl.when(p < _SB // 2 - 1)
                def _():
                    pltpu.async_copy(x_hbm.at[c].at[sidx.at[j0 + 2]],
                                     vals0, gsem0)
                pltpu.make_async_copy(x_hbm.at[c].at[sidx.at[j0 + 1]],
                                      vals1, gsem1).wait()
                pltpu.sync_copy(vals1, accum.at[didx.at[j0 + 1]], add=True)
                return carry2
            lax.fori_loop(0, _SB // 2, pair, 0)
            return carry
        lax.fori_loop(0, nch // _SB, superchunk, 0)
        plsc.subcore_barrier()

        pltpu.sync_copy(accum.at[pl.ds(r0, rows_per_tile)],
                        out_hbm.at[c].at[pl.ds(r0, rows_per_tile)])

    return pl.kernel(
        body,
        out_type=jax.ShapeDtypeStruct((_NC, n_pad, dh), jnp.float32),
        mesh=mesh,
        scratch_types=[
            pltpu.VMEM((_SB, _CHUNK), jnp.int32),
            pltpu.VMEM((_SB, _CHUNK), jnp.int32),
            pltpu.VMEM((_CHUNK, dh), jnp.float32),
            pltpu.VMEM((_CHUNK, dh), jnp.float32),
            pltpu.VMEM_SHARED((n_pad, dh), jnp.float32),
            pltpu.SemaphoreType.DMA,
            pltpu.SemaphoreType.DMA,
        ],
    )


def _sc_linear_segsum(n_pad, dh, nch):
    rows_per_tile = n_pad // _NS
    mesh = plsc.VectorSubcoreMesh(core_axis_name="c", subcore_axis_name="s",
                                  num_cores=_NC)

    def body(val_hbm, dst_hbm, z_hbm, out_hbm, didx, vals, accum, gsem):
        c = lax.axis_index("c")
        s = lax.axis_index("s")
        wid = s * _NC + c
        r0 = s * rows_per_tile

        pltpu.sync_copy(z_hbm, vals)

        def zrow(i, carry):
            pltpu.sync_copy(vals, accum.at[pl.ds(r0 + i * _CHUNK, _CHUNK)])
            return carry
        lax.fori_loop(0, rows_per_tile // _CHUNK, zrow, 0)
        plsc.subcore_barrier()

        def superchunk(i, carry):
            pltpu.sync_copy(dst_hbm.at[wid].at[pl.ds(i * _SB, _SB)], didx)

            def chunk(j, carry2):
                pltpu.async_copy(val_hbm.at[wid, i * _SB + j], vals,
                                 gsem).wait()
                pltpu.sync_copy(vals, accum.at[didx.at[j]], add=True)
                return carry2
            lax.fori_loop(0, _SB, chunk, 0)
            return carry
        lax.fori_loop(0, nch // _SB, superchunk, 0)
        plsc.subcore_barrier()

        pltpu.sync_copy(accum.at[pl.ds(r0, rows_per_tile)],
                        out_hbm.at[c].at[pl.ds(r0, rows_per_tile)])

    return pl.kernel(
        body,
        out_type=jax.ShapeDtypeStruct((_NC, n_pad, dh), jnp.float32),
        mesh=mesh,
        scratch_types=[
            pltpu.VMEM((_SB, _CHUNK), jnp.int32),
            pltpu.VMEM((_CHUNK, dh), jnp.float32),
            pltpu.VMEM_SHARED((n_pad, dh), jnp.float32),
            pltpu.SemaphoreType.DMA,
        ],
    )


def _update_body(xp_ref, g_ref, sd_ref, u1_ref, u2_ref, w3_ref,
                 m1_ref, m2_ref, mb_ref, o_ref):
    f32 = jnp.float32
    x = xp_ref[0]
    g = g_ref[0]
    sd = sd_ref[0]
    deg = sd[:, 16:17]
    msg = (deg * jnp.dot(x, u1_ref[...], preferred_element_type=f32)
           + jnp.dot(g, u2_ref[...], preferred_element_type=f32)
           + jnp.dot(sd, w3_ref[...], preferred_element_type=f32))
    o_ref[0] = jnp.maximum(
        jnp.dot(x, m1_ref[...], preferred_element_type=f32)
        + jnp.dot(msg, m2_ref[...], preferred_element_type=f32)
        + mb_ref[...], 0.0)


def _tc_update(xparts, gparts, sdparts, u1, u2, w3, m1, m2, mb):
    _, n_pad, dh = xparts.shape
    d = dh
    blk = 1024
    grid = n_pad // blk
    full = lambda i: (0, 0)
    half = pl.BlockSpec((_NC, blk, dh), lambda i: (0, i, 0))
    return pl.pallas_call(
        _update_body,
        grid=(grid,),
        in_specs=[
            half,
            half,
            half,
            pl.BlockSpec((d, d), full),
            pl.BlockSpec((d, d), full),
            pl.BlockSpec((dh, d), full),
            pl.BlockSpec((d, d), full),
            pl.BlockSpec((d, d), full),
            pl.BlockSpec((1, d), full),
        ],
        out_specs=half,
        out_shape=jax.ShapeDtypeStruct((_NC, n_pad, dh), jnp.float32),
    )(xparts, gparts, sdparts, u1, u2, w3, m1, m2, mb)


def kernel(x, edge_index, edge_attr, U_W, U_b, M_W, M_b):
    n, d = x.shape
    dh = d
    e = edge_index.shape[1]
    de = edge_attr.shape[1]
    t = U_W.shape[0]

    sbe = _CHUNK * _SB
    ept = -(-e // (_NS * sbe)) * sbe
    nch = ept // _CHUNK
    ep = ept * _NS
    pad = ep - e
    n_pad = -(-(n + 16) // (_CHUNK * _NS)) * (_CHUNK * _NS)
    junk = n_pad - n

    src = edge_index[0]
    dst = edge_index[1]
    ar = jnp.arange(pad, dtype=jnp.int32)
    src_p = jnp.concatenate([src, ar % n]).reshape(_NS, nch, _CHUNK)
    dst_p = jnp.concatenate([dst, n + ar % junk]).reshape(_NS, nch, _CHUNK)
    dst_p32 = dst_p.reshape(_NC * _NS, nch // _NC, _CHUNK)

    aug = jnp.concatenate(
        [edge_attr, jnp.ones((e, 1), jnp.float32),
         jnp.zeros((e, dh - de - 1), jnp.float32)], axis=1)
    aug_p = jnp.pad(aug, ((0, pad), (0, 0))).reshape(
        _NC * _NS, nch // _NC, _CHUNK, dh)

    xp = jnp.pad(x, ((0, junk), (0, 0)))
    xparts = xp[None]
    zeros = jnp.zeros((_CHUNK, dh), jnp.float32)

    seg_gather = _sc_gather_segsum(n_pad, dh, nch)
    seg_linear = _sc_linear_segsum(n_pad, dh, nch // _NC)

    sdparts = seg_linear(aug_p, dst_p32, zeros)

    for k in range(t):
        u1 = U_W[k, :d]
        u2 = U_W[k, d:2 * d]
        w3 = jnp.zeros((dh, d), jnp.float32)
        w3 = w3.at[:de].set(U_W[k, 2 * d:]).at[de].set(U_b[k])
        m1 = M_W[k, :d]
        m2 = M_W[k, d:]
        mb = M_b[k][None, :]
        gparts = seg_gather(xparts, src_p, dst_p, zeros)
        xparts = _tc_update(xparts, gparts, sdparts, u1, u2, w3, m1, m2, mb)
    return xparts[0][:n]


# speedup vs baseline: 6.2788x; 1.0436x over previous
import jax
import jax.numpy as jnp
from jax import lax
from jax.experimental import pallas as pl
from jax.experimental.pallas import tpu as pltpu
from jax.experimental.pallas import tpu_sc as plsc

_NC = 2
_NS = 16
_RT = _NC * _NS
_L = 16
_CHUNK = 128
_SB = 40
_JR = 256


def _sc_route(nchr, cap_ch, half):
    cap = cap_ch * _CHUNK
    mesh = plsc.VectorSubcoreMesh(core_axis_name="c", subcore_axis_name="s",
                                  num_cores=_NC)

    def body(src_hbm, dst_hbm, lists_hbm, cnt_hbm,
             sidx, didx, l0s, l0d, l1s, l1d, cbuf):
        c = lax.axis_index("c")
        s = lax.axis_index("s")
        rt = s * _NC + c
        iota = lax.iota(jnp.int32, _L)

        def superchunk(i, off):
            pltpu.sync_copy(src_hbm.at[rt].at[pl.ds(i * _SB, _SB)], sidx)
            pltpu.sync_copy(dst_hbm.at[rt].at[pl.ds(i * _SB, _SB)], didx)

            def chunk(j, off2):
                o0, o1 = off2
                for v in range(_CHUNK // _L):
                    sv = sidx[j, pl.ds(v * _L, _L)]
                    dv = didx[j, pl.ds(v * _L, _L)]
                    m0 = dv < half
                    m1 = jnp.logical_not(m0)
                    p0 = o0 + plsc.cumsum(m0.astype(jnp.int32)) - 1
                    p1 = o1 + plsc.cumsum(m1.astype(jnp.int32)) - 1
                    plsc.store_scatter(l0s, [p0], sv, mask=m0)
                    plsc.store_scatter(l0d, [p0], dv, mask=m0)
                    plsc.store_scatter(l1s, [p1], sv, mask=m1)
                    plsc.store_scatter(l1d, [p1], dv - half, mask=m1)
                    n0 = jnp.max(plsc.all_reduce_population_count(m0))
                    o0 = o0 + n0
                    o1 = o1 + (_L - n0)
                return (o0, o1)
            return lax.fori_loop(0, _SB, chunk, off)
        o0, o1 = lax.fori_loop(0, nchr // _SB, superchunk,
                               (jnp.int32(0), jnp.int32(0)))

        for v in range(_CHUNK // _L):
            jsrc = (iota + v * _L + rt * 97) % half
            jdst = half + ((iota + v * _L) % _JR)
            l0s[pl.ds(o0 + v * _L, _L)] = jsrc
            l0d[pl.ds(o0 + v * _L, _L)] = jdst
            l1s[pl.ds(o1 + v * _L, _L)] = jsrc
            l1d[pl.ds(o1 + v * _L, _L)] = jdst
        n0 = (o0 + _CHUNK - 1) // _CHUNK
        n1 = (o1 + _CHUNK - 1) // _CHUNK

        for v in range(_CHUNK // _L):
            cbuf[0, pl.ds(v * _L, _L)] = jnp.full((_L,), n0, jnp.int32)
            cbuf[1, pl.ds(v * _L, _L)] = jnp.full((_L,), n1, jnp.int32)
            for r in range(2, 8):
                cbuf[r, pl.ds(v * _L, _L)] = jnp.zeros((_L,), jnp.int32)
        pltpu.sync_copy(cbuf, cnt_hbm.at[rt])

        base = rt * 4 * cap
        pltpu.sync_copy(l0s, lists_hbm.at[pl.ds(base, cap)])
        pltpu.sync_copy(l0d, lists_hbm.at[pl.ds(base + cap, cap)])
        pltpu.sync_copy(l1s, lists_hbm.at[pl.ds(base + 2 * cap, cap)])
        pltpu.sync_copy(l1d, lists_hbm.at[pl.ds(base + 3 * cap, cap)])

    return pl.kernel(
        body,
        out_type=(jax.ShapeDtypeStruct((_RT * 4 * cap,), jnp.int32),
                  jax.ShapeDtypeStruct((_RT, 8, _CHUNK), jnp.int32)),
        mesh=mesh,
        compiler_params=pltpu.CompilerParams(needs_layout_passes=False),
        scratch_types=[
            pltpu.VMEM((_SB, _CHUNK), jnp.int32),
            pltpu.VMEM((_SB, _CHUNK), jnp.int32),
            pltpu.VMEM((cap,), jnp.int32),
            pltpu.VMEM((cap,), jnp.int32),
            pltpu.VMEM((cap,), jnp.int32),
            pltpu.VMEM((cap,), jnp.int32),
            pltpu.VMEM((8, _CHUNK), jnp.int32),
        ],
    )


def _sc_gather_routed(n_pad, d, cap_ch, half):
    cap = cap_ch * _CHUNK
    arows = half + _JR
    rows_per_tile = arows // _NS
    out_rows_per_tile = half // _NS
    mesh = plsc.VectorSubcoreMesh(core_axis_name="c", subcore_axis_name="s",
                                  num_cores=_NC)

    def body(x_hbm, lists_hbm, cnt_hbm, z_hbm, out_hbm,
             i0s, i0d, i1s, i1d, vals0, vals1, cbuf, accum,
             gsem0, gsem1, isem0, isem1):
        c = lax.axis_index("c")
        s = lax.axis_index("s")
        r0 = s * rows_per_tile

        pltpu.sync_copy(z_hbm, vals0)
        nzfull = rows_per_tile // _CHUNK
        rem = rows_per_tile - nzfull * _CHUNK

        def zrow(i, carry):
            pltpu.sync_copy(vals0, accum.at[pl.ds(r0 + i * _CHUNK, _CHUNK)])
            return carry
        lax.fori_loop(0, nzfull, zrow, 0)
        if rem:
            pltpu.sync_copy(vals0.at[pl.ds(0, rem)],
                            accum.at[pl.ds(r0 + nzfull * _CHUNK, rem)])
        plsc.subcore_barrier()

        rt0 = 2 * s
        rt1 = 2 * s + 1
        pltpu.sync_copy(cnt_hbm.at[rt0], cbuf)
        n0 = jnp.max(jnp.where(c == 0, cbuf[0, pl.ds(0, _L)],
                               cbuf[1, pl.ds(0, _L)]))
        pltpu.sync_copy(cnt_hbm.at[rt1], cbuf)
        n1 = jnp.max(jnp.where(c == 0, cbuf[0, pl.ds(0, _L)],
                               cbuf[1, pl.ds(0, _L)]))
        nt = n0 + n1

        def src_off(i):
            rt = jnp.where(i < n0, rt0, rt1)
            j = jnp.where(i < n0, i, i - n0)
            return (rt * 4 + 2 * c) * cap + j * _CHUNK

        def stage(i, isref, idref, isem):
            off = src_off(i)
            pltpu.async_copy(lists_hbm.at[pl.ds(off, _CHUNK)], isref, isem)
            pltpu.async_copy(lists_hbm.at[pl.ds(off + cap, _CHUNK)],
                             idref, isem)

        def wait_idx(i, isref, idref, isem):
            off = src_off(i)
            pltpu.make_async_copy(lists_hbm.at[pl.ds(off, _CHUNK)],
                                  isref, isem).wait()
            pltpu.make_async_copy(lists_hbm.at[pl.ds(off + cap, _CHUNK)],
                                  idref, isem).wait()

        @pl.when(nt > 0)
        def _():
            stage(0, i0s, i0d, isem0)
            wait_idx(0, i0s, i0d, isem0)
            pltpu.async_copy(x_hbm.at[i0s], vals0, gsem0)

        @pl.when(nt > 1)
        def _():
            stage(1, i1s, i1d, isem1)

        def pair(p, carry):
            j0 = 2 * p
            j1 = j0 + 1

            @pl.when(j1 < nt)
            def _():
                wait_idx(j1, i1s, i1d, isem1)
                pltpu.async_copy(x_hbm.at[i1s], vals1, gsem1)
            pltpu.make_async_copy(x_hbm.at[i0s], vals0, gsem0).wait()
            pltpu.sync_copy(vals0, accum.at[i0d], add=True)

            @pl.when(j0 + 2 < nt)
            def _():
                stage(j0 + 2, i0s, i0d, isem0)
                wait_idx(j0 + 2, i0s, i0d, isem0)
                pltpu.async_copy(x_hbm.at[i0s], vals0, gsem0)

            @pl.when(j1 < nt)
            def _():
                pltpu.make_async_copy(x_hbm.at[i1s], vals1, gsem1).wait()
                pltpu.sync_copy(vals1, accum.at[i1d], add=True)

            @pl.when(j1 + 2 < nt)
            def _():
                stage(j1 + 2, i1s, i1d, isem1)
            return carry
        lax.fori_loop(0, (nt + 1) // 2, pair, 0)
        plsc.subcore_barrier()

        pltpu.sync_copy(
            accum.at[pl.ds(s * out_rows_per_tile, out_rows_per_tile)],
            out_hbm.at[pl.ds(c * half + s * out_rows_per_tile,
                             out_rows_per_tile)])

    return pl.kernel(
        body,
        out_type=jax.ShapeDtypeStruct((_NC * half, d), jnp.float32),
        mesh=mesh,
        compiler_params=pltpu.CompilerParams(needs_layout_passes=False),
        scratch_types=[
            pltpu.VMEM((_CHUNK,), jnp.int32),
            pltpu.VMEM((_CHUNK,), jnp.int32),
            pltpu.VMEM((_CHUNK,), jnp.int32),
            pltpu.VMEM((_CHUNK,), jnp.int32),
            pltpu.VMEM((_CHUNK, d), jnp.float32),
            pltpu.VMEM((_CHUNK, d), jnp.float32),
            pltpu.VMEM((8, _CHUNK), jnp.int32),
            pltpu.VMEM_SHARED((arows, d), jnp.float32),
            pltpu.SemaphoreType.DMA,
            pltpu.SemaphoreType.DMA,
            pltpu.SemaphoreType.DMA,
            pltpu.SemaphoreType.DMA,
        ],
    )


def _sc_linear_segsum(n_pad, da, nch):
    rows_per_tile = n_pad // _NS
    mesh = plsc.VectorSubcoreMesh(core_axis_name="c", subcore_axis_name="s",
                                  num_cores=1)

    def body(val_hbm, dst_hbm, z_hbm, out_hbm, didx, vals, accum, gsem):
        s_ = lax.axis_index("s")
        wid = s_
        r0 = s_ * rows_per_tile

        pltpu.sync_copy(z_hbm, vals)

        def zrow(i, carry):
            pltpu.sync_copy(vals, accum.at[pl.ds(r0 + i * _CHUNK, _CHUNK)])
            return carry
        lax.fori_loop(0, rows_per_tile // _CHUNK, zrow, 0)
        plsc.subcore_barrier()

        def superchunk(i, carry):
            pltpu.sync_copy(dst_hbm.at[wid].at[pl.ds(i * _SB, _SB)], didx)

            def chunk(j, carry2):
                pltpu.async_copy(val_hbm.at[wid, i * _SB + j], vals,
                                 gsem).wait()
                pltpu.sync_copy(vals, accum.at[didx.at[j]], add=True)
                return carry2
            lax.fori_loop(0, _SB, chunk, 0)
            return carry
        lax.fori_loop(0, nch // _SB, superchunk, 0)
        plsc.subcore_barrier()

        pltpu.sync_copy(accum.at[pl.ds(r0, rows_per_tile)],
                        out_hbm.at[0].at[pl.ds(r0, rows_per_tile)])

    return pl.kernel(
        body,
        out_type=jax.ShapeDtypeStruct((1, n_pad, da), jnp.float32),
        mesh=mesh,
        scratch_types=[
            pltpu.VMEM((_SB, _CHUNK), jnp.int32),
            pltpu.VMEM((_CHUNK, da), jnp.float32),
            pltpu.VMEM_SHARED((n_pad, da), jnp.float32),
            pltpu.SemaphoreType.DMA,
        ],
    )


def _update_body(x_ref, g_ref, sd_ref, u1_ref, u2_ref, w3_ref,
                 m1_ref, m2_ref, mb_ref, o_ref):
    f32 = jnp.float32
    x = x_ref[...]
    g = g_ref[...]
    sd = sd_ref[0]
    deg = sd[:, 16:17]
    msg = (deg * jnp.dot(x, u1_ref[...], preferred_element_type=f32)
           + jnp.dot(g, u2_ref[...], preferred_element_type=f32)
           + jnp.dot(sd, w3_ref[...], preferred_element_type=f32))
    o_ref[...] = jnp.maximum(
        jnp.dot(x, m1_ref[...], preferred_element_type=f32)
        + jnp.dot(msg, m2_ref[...], preferred_element_type=f32)
        + mb_ref[...], 0.0)


def _tc_update(x, g, sdparts, u1, u2, w3, m1, m2, mb):
    n_pad, d = x.shape
    da = sdparts.shape[2]
    blk = 1024
    grid = n_pad // blk
    full = lambda i: (0, 0)
    row = pl.BlockSpec((blk, d), lambda i: (i, 0))
    return pl.pallas_call(
        _update_body,
        grid=(grid,),
        in_specs=[
            row,
            row,
            pl.BlockSpec((1, blk, da), lambda i: (0, i, 0)),
            pl.BlockSpec((d, d), full),
            pl.BlockSpec((d, d), full),
            pl.BlockSpec((da, d), full),
            pl.BlockSpec((d, d), full),
            pl.BlockSpec((d, d), full),
            pl.BlockSpec((1, d), full),
        ],
        out_specs=row,
        out_shape=jax.ShapeDtypeStruct((n_pad, d), jnp.float32),
    )(x, g, sdparts, u1, u2, w3, m1, m2, mb)


def kernel(x, edge_index, edge_attr, U_W, U_b, M_W, M_b):
    n, d = x.shape
    e = edge_index.shape[1]
    de = edge_attr.shape[1]
    t = U_W.shape[0]

    sbe = _CHUNK * _SB
    eptr = -(-e // (_RT * sbe)) * sbe
    nchr = eptr // _CHUNK
    ep = eptr * _RT
    pad = ep - e
    n_pad = -(-(n + 16) // (_CHUNK * _NS)) * (_CHUNK * _NS)
    junk = n_pad - n
    half = n_pad // _NC
    cap_ch = nchr + 1

    src = edge_index[0]
    dst = edge_index[1]
    ar = jnp.arange(pad, dtype=jnp.int32)
    src_p = jnp.concatenate([src, ar % n]).reshape(_RT, nchr, _CHUNK)
    dst_p = jnp.concatenate([dst, n + ar % junk]).reshape(_RT, nchr, _CHUNK)

    da = d
    aug = jnp.concatenate(
        [edge_attr, jnp.ones((e, 1), jnp.float32),
         jnp.zeros((e, da - de - 1), jnp.float32)], axis=1)
    aug_p = jnp.pad(aug, ((0, pad), (0, 0))).reshape(
        _NS, 2 * nchr, _CHUNK, da)
    dst_p16 = dst_p.reshape(_NS, 2 * nchr, _CHUNK)

    xp = jnp.pad(x, ((0, junk), (0, 0)))
    zeros = jnp.zeros((_CHUNK, d), jnp.float32)
    zeros_sa = jnp.zeros((_CHUNK, da), jnp.float32)

    lists, cnts = _sc_route(nchr, cap_ch, half)(src_p, dst_p)
    seg_gather = _sc_gather_routed(n_pad, d, cap_ch, half)
    sdparts = _sc_linear_segsum(n_pad, da, 2 * nchr)(aug_p, dst_p16, zeros_sa)

    for k in range(t):
        u1 = U_W[k, :d]
        u2 = U_W[k, d:2 * d]
        w3 = jnp.zeros((da, d), jnp.float32)
        w3 = w3.at[:de].set(U_W[k, 2 * d:]).at[de].set(U_b[k])
        m1 = M_W[k, :d]
        m2 = M_W[k, d:]
        mb = M_b[k][None, :]
        g = seg_gather(xp, lists, cnts, zeros)
        xp = _tc_update(xp, g, sdparts, u1, u2, w3, m1, m2, mb)
    return xp[:n]


# speedup vs baseline: 7.4737x; 1.1903x over previous
import jax
import jax.numpy as jnp
from jax import lax
from jax.experimental import pallas as pl
from jax.experimental.pallas import tpu as pltpu
from jax.experimental.pallas import tpu_sc as plsc

_NC = 2
_NS = 16
_RT = _NC * _NS
_L = 16
_CHUNK = 128
_SB = 40
_JR = 256


def _sc_route(nchr, cap_ch, half):
    cap = cap_ch * _CHUNK
    mesh = plsc.VectorSubcoreMesh(core_axis_name="c", subcore_axis_name="s",
                                  num_cores=_NC)

    def body(src_hbm, dst_hbm, lists_hbm, cnt_hbm,
             sidx, didx, l0s, l0d, l0e, l1s, l1d, l1e, cbuf):
        c = lax.axis_index("c")
        s = lax.axis_index("s")
        rt = s * _NC + c
        iota = lax.iota(jnp.int32, _L)

        ept = nchr * _CHUNK

        def superchunk(i, off):
            pltpu.sync_copy(src_hbm.at[rt].at[pl.ds(i * _SB, _SB)], sidx)
            pltpu.sync_copy(dst_hbm.at[rt].at[pl.ds(i * _SB, _SB)], didx)

            def chunk(j, off2):
                o0, o1 = off2
                ebase = rt * ept + (i * _SB + j) * _CHUNK
                for v in range(_CHUNK // _L):
                    sv = sidx[j, pl.ds(v * _L, _L)]
                    dv = didx[j, pl.ds(v * _L, _L)]
                    ev = ebase + v * _L + iota
                    m0 = dv < half
                    m1 = jnp.logical_not(m0)
                    p0 = o0 + plsc.cumsum(m0.astype(jnp.int32)) - 1
                    p1 = o1 + plsc.cumsum(m1.astype(jnp.int32)) - 1
                    plsc.store_scatter(l0s, [p0], sv, mask=m0)
                    plsc.store_scatter(l0d, [p0], dv, mask=m0)
                    plsc.store_scatter(l0e, [p0], ev, mask=m0)
                    plsc.store_scatter(l1s, [p1], sv, mask=m1)
                    plsc.store_scatter(l1d, [p1], dv - half, mask=m1)
                    plsc.store_scatter(l1e, [p1], ev, mask=m1)
                    n0 = jnp.max(plsc.all_reduce_population_count(m0))
                    o0 = o0 + n0
                    o1 = o1 + (_L - n0)
                return (o0, o1)
            return lax.fori_loop(0, _SB, chunk, off)
        o0, o1 = lax.fori_loop(0, nchr // _SB, superchunk,
                               (jnp.int32(0), jnp.int32(0)))

        for v in range(_CHUNK // _L):
            jsrc = (iota + v * _L + rt * 97) % half
            jdst = half + ((iota + v * _L) % _JR)
            jeid = (iota + v * _L + rt * 131) % ept
            l0s[pl.ds(o0 + v * _L, _L)] = jsrc
            l0d[pl.ds(o0 + v * _L, _L)] = jdst
            l0e[pl.ds(o0 + v * _L, _L)] = jeid
            l1s[pl.ds(o1 + v * _L, _L)] = jsrc
            l1d[pl.ds(o1 + v * _L, _L)] = jdst
            l1e[pl.ds(o1 + v * _L, _L)] = jeid
        n0 = (o0 + _CHUNK - 1) // _CHUNK
        n1 = (o1 + _CHUNK - 1) // _CHUNK

        for v in range(_CHUNK // _L):
            cbuf[0, pl.ds(v * _L, _L)] = jnp.full((_L,), n0, jnp.int32)
            cbuf[1, pl.ds(v * _L, _L)] = jnp.full((_L,), n1, jnp.int32)
            for r in range(2, 8):
                cbuf[r, pl.ds(v * _L, _L)] = jnp.zeros((_L,), jnp.int32)
        pltpu.sync_copy(cbuf, cnt_hbm.at[rt])

        base = rt * 6 * cap
        pltpu.sync_copy(l0s, lists_hbm.at[pl.ds(base, cap)])
        pltpu.sync_copy(l0d, lists_hbm.at[pl.ds(base + cap, cap)])
        pltpu.sync_copy(l0e, lists_hbm.at[pl.ds(base + 2 * cap, cap)])
        pltpu.sync_copy(l1s, lists_hbm.at[pl.ds(base + 3 * cap, cap)])
        pltpu.sync_copy(l1d, lists_hbm.at[pl.ds(base + 4 * cap, cap)])
        pltpu.sync_copy(l1e, lists_hbm.at[pl.ds(base + 5 * cap, cap)])

    return pl.kernel(
        body,
        out_type=(jax.ShapeDtypeStruct((_RT * 6 * cap,), jnp.int32),
                  jax.ShapeDtypeStruct((_RT, 8, _CHUNK), jnp.int32)),
        mesh=mesh,
        compiler_params=pltpu.CompilerParams(needs_layout_passes=False),
        scratch_types=[
            pltpu.VMEM((_SB, _CHUNK), jnp.int32),
            pltpu.VMEM((_SB, _CHUNK), jnp.int32),
            pltpu.VMEM((cap,), jnp.int32),
            pltpu.VMEM((cap,), jnp.int32),
            pltpu.VMEM((cap,), jnp.int32),
            pltpu.VMEM((cap,), jnp.int32),
            pltpu.VMEM((cap,), jnp.int32),
            pltpu.VMEM((cap,), jnp.int32),
            pltpu.VMEM((8, _CHUNK), jnp.int32),
        ],
    )


def _sc_gather_routed(n_pad, d, cap_ch, half, gsel):
    cap = cap_ch * _CHUNK
    arows = half + _JR
    rows_per_tile = arows // _NS
    out_rows_per_tile = half // _NS
    mesh = plsc.VectorSubcoreMesh(core_axis_name="c", subcore_axis_name="s",
                                  num_cores=_NC)

    def body(x_hbm, lists_hbm, cnt_hbm, z_hbm, out_hbm,
             i0s, i0d, i1s, i1d, vals0, vals1, cbuf, accum,
             gsem0, gsem1, isem0, isem1):
        c = lax.axis_index("c")
        s = lax.axis_index("s")
        r0 = s * rows_per_tile

        pltpu.sync_copy(z_hbm, vals0)
        nzfull = rows_per_tile // _CHUNK
        rem = rows_per_tile - nzfull * _CHUNK

        def zrow(i, carry):
            pltpu.sync_copy(vals0, accum.at[pl.ds(r0 + i * _CHUNK, _CHUNK)])
            return carry
        lax.fori_loop(0, nzfull, zrow, 0)
        if rem:
            pltpu.sync_copy(vals0.at[pl.ds(0, rem)],
                            accum.at[pl.ds(r0 + nzfull * _CHUNK, rem)])
        plsc.subcore_barrier()

        rt0 = 2 * s
        rt1 = 2 * s + 1
        pltpu.sync_copy(cnt_hbm.at[rt0], cbuf)
        n0 = jnp.max(jnp.where(c == 0, cbuf[0, pl.ds(0, _L)],
                               cbuf[1, pl.ds(0, _L)]))
        pltpu.sync_copy(cnt_hbm.at[rt1], cbuf)
        n1 = jnp.max(jnp.where(c == 0, cbuf[0, pl.ds(0, _L)],
                               cbuf[1, pl.ds(0, _L)]))
        nt = n0 + n1

        def src_off(i):
            rt = jnp.where(i < n0, rt0, rt1)
            j = jnp.where(i < n0, i, i - n0)
            return (rt * 6 + 3 * c) * cap + j * _CHUNK

        def stage(i, isref, idref, isem):
            off = src_off(i)
            pltpu.async_copy(lists_hbm.at[pl.ds(off + gsel * cap, _CHUNK)],
                             isref, isem)
            pltpu.async_copy(lists_hbm.at[pl.ds(off + cap, _CHUNK)],
                             idref, isem)

        def wait_idx(i, isref, idref, isem):
            off = src_off(i)
            pltpu.make_async_copy(
                lists_hbm.at[pl.ds(off + gsel * cap, _CHUNK)],
                isref, isem).wait()
            pltpu.make_async_copy(lists_hbm.at[pl.ds(off + cap, _CHUNK)],
                                  idref, isem).wait()

        @pl.when(nt > 0)
        def _():
            stage(0, i0s, i0d, isem0)
            wait_idx(0, i0s, i0d, isem0)
            pltpu.async_copy(x_hbm.at[i0s], vals0, gsem0)

        @pl.when(nt > 1)
        def _():
            stage(1, i1s, i1d, isem1)

        def pair(p, carry):
            j0 = 2 * p
            j1 = j0 + 1

            @pl.when(j1 < nt)
            def _():
                wait_idx(j1, i1s, i1d, isem1)
                pltpu.async_copy(x_hbm.at[i1s], vals1, gsem1)
            pltpu.make_async_copy(x_hbm.at[i0s], vals0, gsem0).wait()
            pltpu.sync_copy(vals0, accum.at[i0d], add=True)

            @pl.when(j0 + 2 < nt)
            def _():
                stage(j0 + 2, i0s, i0d, isem0)
                wait_idx(j0 + 2, i0s, i0d, isem0)
                pltpu.async_copy(x_hbm.at[i0s], vals0, gsem0)

            @pl.when(j1 < nt)
            def _():
                pltpu.make_async_copy(x_hbm.at[i1s], vals1, gsem1).wait()
                pltpu.sync_copy(vals1, accum.at[i1d], add=True)

            @pl.when(j1 + 2 < nt)
            def _():
                stage(j1 + 2, i1s, i1d, isem1)
            return carry
        lax.fori_loop(0, (nt + 1) // 2, pair, 0)
        plsc.subcore_barrier()

        pltpu.sync_copy(
            accum.at[pl.ds(s * out_rows_per_tile, out_rows_per_tile)],
            out_hbm.at[pl.ds(c * half + s * out_rows_per_tile,
                             out_rows_per_tile)])

    return pl.kernel(
        body,
        out_type=jax.ShapeDtypeStruct((_NC * half, d), jnp.float32),
        mesh=mesh,
        compiler_params=pltpu.CompilerParams(needs_layout_passes=False),
        scratch_types=[
            pltpu.VMEM((_CHUNK,), jnp.int32),
            pltpu.VMEM((_CHUNK,), jnp.int32),
            pltpu.VMEM((_CHUNK,), jnp.int32),
            pltpu.VMEM((_CHUNK,), jnp.int32),
            pltpu.VMEM((_CHUNK, d), jnp.float32),
            pltpu.VMEM((_CHUNK, d), jnp.float32),
            pltpu.VMEM((8, _CHUNK), jnp.int32),
            pltpu.VMEM_SHARED((arows, d), jnp.float32),
            pltpu.SemaphoreType.DMA,
            pltpu.SemaphoreType.DMA,
            pltpu.SemaphoreType.DMA,
            pltpu.SemaphoreType.DMA,
        ],
    )


def _update_body(x_ref, g_ref, sd_ref, u1_ref, u2_ref, w3_ref,
                 m1_ref, m2_ref, mb_ref, o_ref):
    f32 = jnp.float32
    x = x_ref[...]
    g = g_ref[...]
    sd = sd_ref[...]
    deg = sd[:, 16:17]
    msg = (deg * jnp.dot(x, u1_ref[...], preferred_element_type=f32)
           + jnp.dot(g, u2_ref[...], preferred_element_type=f32)
           + jnp.dot(sd, w3_ref[...], preferred_element_type=f32))
    o_ref[...] = jnp.maximum(
        jnp.dot(x, m1_ref[...], preferred_element_type=f32)
        + jnp.dot(msg, m2_ref[...], preferred_element_type=f32)
        + mb_ref[...], 0.0)


def _tc_update(x, g, sdparts, u1, u2, w3, m1, m2, mb):
    n_pad, d = x.shape
    da = sdparts.shape[1]
    blk = 1024
    grid = n_pad // blk
    full = lambda i: (0, 0)
    row = pl.BlockSpec((blk, d), lambda i: (i, 0))
    return pl.pallas_call(
        _update_body,
        grid=(grid,),
        in_specs=[
            row,
            row,
            pl.BlockSpec((blk, da), lambda i: (i, 0)),
            pl.BlockSpec((d, d), full),
            pl.BlockSpec((d, d), full),
            pl.BlockSpec((da, d), full),
            pl.BlockSpec((d, d), full),
            pl.BlockSpec((d, d), full),
            pl.BlockSpec((1, d), full),
        ],
        out_specs=row,
        out_shape=jax.ShapeDtypeStruct((n_pad, d), jnp.float32),
    )(x, g, sdparts, u1, u2, w3, m1, m2, mb)


def kernel(x, edge_index, edge_attr, U_W, U_b, M_W, M_b):
    n, d = x.shape
    e = edge_index.shape[1]
    de = edge_attr.shape[1]
    t = U_W.shape[0]

    sbe = _CHUNK * _SB
    eptr = -(-e // (_RT * sbe)) * sbe
    nchr = eptr // _CHUNK
    ep = eptr * _RT
    pad = ep - e
    n_pad = -(-(n + 16) // (_CHUNK * _NS)) * (_CHUNK * _NS)
    junk = n_pad - n
    half = n_pad // _NC
    cap_ch = nchr + 1

    src = edge_index[0]
    dst = edge_index[1]
    ar = jnp.arange(pad, dtype=jnp.int32)
    src_p = jnp.concatenate([src, ar % n]).reshape(_RT, nchr, _CHUNK)
    dst_p = jnp.concatenate([dst, n + ar % junk]).reshape(_RT, nchr, _CHUNK)

    da = d
    aug = jnp.concatenate(
        [edge_attr, jnp.ones((e, 1), jnp.float32),
         jnp.zeros((e, da - de - 1), jnp.float32)], axis=1)
    aug_p = jnp.pad(aug, ((0, pad), (0, 0)))

    xp = jnp.pad(x, ((0, junk), (0, 0)))
    zeros = jnp.zeros((_CHUNK, d), jnp.float32)
    zeros_sa = jnp.zeros((_CHUNK, da), jnp.float32)

    lists, cnts = _sc_route(nchr, cap_ch, half)(src_p, dst_p)
    seg_gather = _sc_gather_routed(n_pad, d, cap_ch, half, gsel=0)
    seg_aug = _sc_gather_routed(n_pad, d, cap_ch, half, gsel=2)
    sdparts = seg_aug(aug_p, lists, cnts, zeros_sa)

    for k in range(t):
        u1 = U_W[k, :d]
        u2 = U_W[k, d:2 * d]
        w3 = jnp.zeros((da, d), jnp.float32)
        w3 = w3.at[:de].set(U_W[k, 2 * d:]).at[de].set(U_b[k])
        m1 = M_W[k, :d]
        m2 = M_W[k, d:]
        mb = M_b[k][None, :]
        g = seg_gather(xp, lists, cnts, zeros)
        xp = _tc_update(xp, g, sdparts, u1, u2, w3, m1, m2, mb)
    return xp[:n]


# speedup vs baseline: 8.3396x; 1.1159x over previous
import jax
import jax.numpy as jnp
from jax import lax
from jax.experimental import pallas as pl
from jax.experimental.pallas import tpu as pltpu
from jax.experimental.pallas import tpu_sc as plsc

_NC = 2
_NS = 16
_RT = _NC * _NS
_L = 16
_CHUNK = 128
_SB = 40
_JR = 256


def _sc_route(nchr, cap_ch, half):
    cap = cap_ch * _CHUNK
    mesh = plsc.VectorSubcoreMesh(core_axis_name="c", subcore_axis_name="s",
                                  num_cores=_NC)

    def body(src_hbm, dst_hbm, lists_hbm, cnt_hbm,
             sidx, didx, l0s, l0d, l0e, l1s, l1d, l1e, cbuf):
        c = lax.axis_index("c")
        s = lax.axis_index("s")
        rt = s * _NC + c
        iota = lax.iota(jnp.int32, _L)

        ept = nchr * _CHUNK

        def superchunk(i, off):
            pltpu.sync_copy(src_hbm.at[rt].at[pl.ds(i * _SB, _SB)], sidx)
            pltpu.sync_copy(dst_hbm.at[rt].at[pl.ds(i * _SB, _SB)], didx)

            def chunk(j, off2):
                o0, o1 = off2
                ebase = rt * ept + (i * _SB + j) * _CHUNK
                for v in range(_CHUNK // _L):
                    sv = sidx[j, pl.ds(v * _L, _L)]
                    dv = didx[j, pl.ds(v * _L, _L)]
                    ev = ebase + v * _L + iota
                    m0 = dv < half
                    m1 = jnp.logical_not(m0)
                    p0 = o0 + plsc.cumsum(m0.astype(jnp.int32)) - 1
                    p1 = o1 + plsc.cumsum(m1.astype(jnp.int32)) - 1
                    plsc.store_scatter(l0s, [p0], sv, mask=m0)
                    plsc.store_scatter(l0d, [p0], dv, mask=m0)
                    plsc.store_scatter(l0e, [p0], ev, mask=m0)
                    plsc.store_scatter(l1s, [p1], sv, mask=m1)
                    plsc.store_scatter(l1d, [p1], dv - half, mask=m1)
                    plsc.store_scatter(l1e, [p1], ev, mask=m1)
                    n0 = jnp.max(plsc.all_reduce_population_count(m0))
                    o0 = o0 + n0
                    o1 = o1 + (_L - n0)
                return (o0, o1)
            return lax.fori_loop(0, _SB, chunk, off)
        o0, o1 = lax.fori_loop(0, nchr // _SB, superchunk,
                               (jnp.int32(0), jnp.int32(0)))

        for v in range(_CHUNK // _L):
            jsrc = (iota + v * _L + rt * 97) % half
            jdst = half + ((iota + v * _L) % _JR)
            jeid = (iota + v * _L + rt * 131) % ept
            l0s[pl.ds(o0 + v * _L, _L)] = jsrc
            l0d[pl.ds(o0 + v * _L, _L)] = jdst
            l0e[pl.ds(o0 + v * _L, _L)] = jeid
            l1s[pl.ds(o1 + v * _L, _L)] = jsrc
            l1d[pl.ds(o1 + v * _L, _L)] = jdst
            l1e[pl.ds(o1 + v * _L, _L)] = jeid
        n0 = (o0 + _CHUNK - 1) // _CHUNK
        n1 = (o1 + _CHUNK - 1) // _CHUNK

        for v in range(_CHUNK // _L):
            cbuf[0, pl.ds(v * _L, _L)] = jnp.full((_L,), n0, jnp.int32)
            cbuf[1, pl.ds(v * _L, _L)] = jnp.full((_L,), n1, jnp.int32)
            for r in range(2, 8):
                cbuf[r, pl.ds(v * _L, _L)] = jnp.zeros((_L,), jnp.int32)
        pltpu.sync_copy(cbuf, cnt_hbm.at[rt])

        base = rt * 6 * cap
        pltpu.sync_copy(l0s, lists_hbm.at[pl.ds(base, cap)])
        pltpu.sync_copy(l0d, lists_hbm.at[pl.ds(base + cap, cap)])
        pltpu.sync_copy(l0e, lists_hbm.at[pl.ds(base + 2 * cap, cap)])
        pltpu.sync_copy(l1s, lists_hbm.at[pl.ds(base + 3 * cap, cap)])
        pltpu.sync_copy(l1d, lists_hbm.at[pl.ds(base + 4 * cap, cap)])
        pltpu.sync_copy(l1e, lists_hbm.at[pl.ds(base + 5 * cap, cap)])

    return pl.kernel(
        body,
        out_type=(jax.ShapeDtypeStruct((_RT * 6 * cap,), jnp.int32),
                  jax.ShapeDtypeStruct((_RT, 8, _CHUNK), jnp.int32)),
        mesh=mesh,
        compiler_params=pltpu.CompilerParams(needs_layout_passes=False),
        scratch_types=[
            pltpu.VMEM((_SB, _CHUNK), jnp.int32),
            pltpu.VMEM((_SB, _CHUNK), jnp.int32),
            pltpu.VMEM((cap,), jnp.int32),
            pltpu.VMEM((cap,), jnp.int32),
            pltpu.VMEM((cap,), jnp.int32),
            pltpu.VMEM((cap,), jnp.int32),
            pltpu.VMEM((cap,), jnp.int32),
            pltpu.VMEM((cap,), jnp.int32),
            pltpu.VMEM((8, _CHUNK), jnp.int32),
        ],
    )


def _sc_gather_routed(n_pad, d, cap_ch, half, gsel):
    cap = cap_ch * _CHUNK
    arows = half + _JR
    rows_per_tile = arows // _NS
    out_rows_per_tile = half // _NS
    mesh = plsc.VectorSubcoreMesh(core_axis_name="c", subcore_axis_name="s",
                                  num_cores=_NC)

    def body(x_hbm, lists_hbm, cnt_hbm, z_hbm, out_hbm,
             ia_s, ia_d, ib_s, ib_d, ic_s, ic_d, id_s, id_d,
             vals0, vals1, cbuf, accum,
             gsem0, gsem1, sema, semb, semc, semd):
        c = lax.axis_index("c")
        s = lax.axis_index("s")
        r0 = s * rows_per_tile

        pltpu.sync_copy(z_hbm, vals0)
        nzfull = rows_per_tile // _CHUNK
        rem = rows_per_tile - nzfull * _CHUNK

        def zrow(i, carry):
            pltpu.sync_copy(vals0, accum.at[pl.ds(r0 + i * _CHUNK, _CHUNK)])
            return carry
        lax.fori_loop(0, nzfull, zrow, 0)
        if rem:
            pltpu.sync_copy(vals0.at[pl.ds(0, rem)],
                            accum.at[pl.ds(r0 + nzfull * _CHUNK, rem)])
        plsc.subcore_barrier()

        rt0 = 2 * s
        rt1 = 2 * s + 1
        pltpu.sync_copy(cnt_hbm.at[rt0], cbuf)
        n0 = jnp.max(jnp.where(c == 0, cbuf[0, pl.ds(0, _L)],
                               cbuf[1, pl.ds(0, _L)]))
        pltpu.sync_copy(cnt_hbm.at[rt1], cbuf)
        n1 = jnp.max(jnp.where(c == 0, cbuf[0, pl.ds(0, _L)],
                               cbuf[1, pl.ds(0, _L)]))
        nt = n0 + n1

        def src_off(i):
            rt = jnp.where(i < n0, rt0, rt1)
            j = jnp.where(i < n0, i, i - n0)
            return (rt * 6 + 3 * c) * cap + j * _CHUNK

        def stage(i, isref, idref, isem):
            off = src_off(i)
            pltpu.async_copy(lists_hbm.at[pl.ds(off + gsel * cap, _CHUNK)],
                             isref, isem)
            pltpu.async_copy(lists_hbm.at[pl.ds(off + cap, _CHUNK)],
                             idref, isem)

        def wait_idx(i, isref, idref, isem):
            off = src_off(i)
            pltpu.make_async_copy(
                lists_hbm.at[pl.ds(off + gsel * cap, _CHUNK)],
                isref, isem).wait()
            pltpu.make_async_copy(lists_hbm.at[pl.ds(off + cap, _CHUNK)],
                                  idref, isem).wait()

        islots = ((ia_s, ia_d, sema), (ib_s, ib_d, semb),
                  (ic_s, ic_d, semc), (id_s, id_d, semd))
        vslots = ((vals0, gsem0), (vals1, gsem1))

        for b in range(4):
            @pl.when(b < nt)
            def _(b=b):
                stage(b, *islots[b])
        for b in range(2):
            @pl.when(b < nt)
            def _(b=b):
                wait_idx(b, *islots[b])
                pltpu.async_copy(x_hbm.at[islots[b][0]], vslots[b][0],
                                 vslots[b][1])

        def quad(q, carry):
            j = 4 * q
            for b in range(4):
                jb = j + b
                v, gs = vslots[b % 2]
                isl = islots[b]
                inx = islots[(b + 2) % 4]

                @pl.when(jb < nt)
                def _(jb=jb, v=v, gs=gs, isl=isl, inx=inx):
                    pltpu.make_async_copy(x_hbm.at[isl[0]], v, gs).wait()
                    pltpu.sync_copy(v, accum.at[isl[1]], add=True)

                    @pl.when(jb + 4 < nt)
                    def _():
                        stage(jb + 4, *isl)

                    @pl.when(jb + 2 < nt)
                    def _():
                        wait_idx(jb + 2, *inx)
                        pltpu.async_copy(x_hbm.at[inx[0]], v, gs)
            return carry
        lax.fori_loop(0, (nt + 3) // 4, quad, 0)
        plsc.subcore_barrier()

        pltpu.sync_copy(
            accum.at[pl.ds(s * out_rows_per_tile, out_rows_per_tile)],
            out_hbm.at[pl.ds(c * half + s * out_rows_per_tile,
                             out_rows_per_tile)])

    return pl.kernel(
        body,
        out_type=jax.ShapeDtypeStruct((_NC * half, d), jnp.float32),
        mesh=mesh,
        compiler_params=pltpu.CompilerParams(needs_layout_passes=False),
        scratch_types=[
            pltpu.VMEM((_CHUNK,), jnp.int32),
            pltpu.VMEM((_CHUNK,), jnp.int32),
            pltpu.VMEM((_CHUNK,), jnp.int32),
            pltpu.VMEM((_CHUNK,), jnp.int32),
            pltpu.VMEM((_CHUNK,), jnp.int32),
            pltpu.VMEM((_CHUNK,), jnp.int32),
            pltpu.VMEM((_CHUNK,), jnp.int32),
            pltpu.VMEM((_CHUNK,), jnp.int32),
            pltpu.VMEM((_CHUNK, d), jnp.float32),
            pltpu.VMEM((_CHUNK, d), jnp.float32),
            pltpu.VMEM((8, _CHUNK), jnp.int32),
            pltpu.VMEM_SHARED((arows, d), jnp.float32),
            pltpu.SemaphoreType.DMA,
            pltpu.SemaphoreType.DMA,
            pltpu.SemaphoreType.DMA,
            pltpu.SemaphoreType.DMA,
            pltpu.SemaphoreType.DMA,
            pltpu.SemaphoreType.DMA,
        ],
    )


def _update_body(x_ref, g_ref, sd_ref, u1_ref, u2_ref, w3_ref,
                 m1_ref, m2_ref, mb_ref, o_ref):
    f32 = jnp.float32
    x = x_ref[...]
    g = g_ref[...]
    sd = sd_ref[...]
    deg = sd[:, 16:17]
    msg = (deg * jnp.dot(x, u1_ref[...], preferred_element_type=f32)
           + jnp.dot(g, u2_ref[...], preferred_element_type=f32)
           + jnp.dot(sd, w3_ref[...], preferred_element_type=f32))
    o_ref[...] = jnp.maximum(
        jnp.dot(x, m1_ref[...], preferred_element_type=f32)
        + jnp.dot(msg, m2_ref[...], preferred_element_type=f32)
        + mb_ref[...], 0.0)


def _tc_update(x, g, sdparts, u1, u2, w3, m1, m2, mb):
    n_pad, d = x.shape
    da = sdparts.shape[1]
    blk = 1024
    grid = n_pad // blk
    full = lambda i: (0, 0)
    row = pl.BlockSpec((blk, d), lambda i: (i, 0))
    return pl.pallas_call(
        _update_body,
        grid=(grid,),
        in_specs=[
            row,
            row,
            pl.BlockSpec((blk, da), lambda i: (i, 0)),
            pl.BlockSpec((d, d), full),
            pl.BlockSpec((d, d), full),
            pl.BlockSpec((da, d), full),
            pl.BlockSpec((d, d), full),
            pl.BlockSpec((d, d), full),
            pl.BlockSpec((1, d), full),
        ],
        out_specs=row,
        out_shape=jax.ShapeDtypeStruct((n_pad, d), jnp.float32),
    )(x, g, sdparts, u1, u2, w3, m1, m2, mb)


def kernel(x, edge_index, edge_attr, U_W, U_b, M_W, M_b):
    n, d = x.shape
    e = edge_index.shape[1]
    de = edge_attr.shape[1]
    t = U_W.shape[0]

    sbe = _CHUNK * _SB
    eptr = -(-e // (_RT * sbe)) * sbe
    nchr = eptr // _CHUNK
    ep = eptr * _RT
    pad = ep - e
    n_pad = -(-(n + 16) // (_CHUNK * _NS)) * (_CHUNK * _NS)
    junk = n_pad - n
    half = n_pad // _NC
    cap_ch = nchr + 1

    src = edge_index[0]
    dst = edge_index[1]
    ar = jnp.arange(pad, dtype=jnp.int32)
    src_p = jnp.concatenate([src, ar % n]).reshape(_RT, nchr, _CHUNK)
    dst_p = jnp.concatenate([dst, n + ar % junk]).reshape(_RT, nchr, _CHUNK)

    da = d
    aug = jnp.concatenate(
        [edge_attr, jnp.ones((e, 1), jnp.float32),
         jnp.zeros((e, da - de - 1), jnp.float32)], axis=1)
    aug_p = jnp.pad(aug, ((0, pad), (0, 0)))

    xp = jnp.pad(x, ((0, junk), (0, 0)))
    zeros = jnp.zeros((_CHUNK, d), jnp.float32)
    zeros_sa = jnp.zeros((_CHUNK, da), jnp.float32)

    lists, cnts = _sc_route(nchr, cap_ch, half)(src_p, dst_p)
    seg_gather = _sc_gather_routed(n_pad, d, cap_ch, half, gsel=0)
    seg_aug = _sc_gather_routed(n_pad, d, cap_ch, half, gsel=2)
    sdparts = seg_aug(aug_p, lists, cnts, zeros_sa)

    for k in range(t):
        u1 = U_W[k, :d]
        u2 = U_W[k, d:2 * d]
        w3 = jnp.zeros((da, d), jnp.float32)
        w3 = w3.at[:de].set(U_W[k, 2 * d:]).at[de].set(U_b[k])
        m1 = M_W[k, :d]
        m2 = M_W[k, d:]
        mb = M_b[k][None, :]
        g = seg_gather(xp, lists, cnts, zeros)
        xp = _tc_update(xp, g, sdparts, u1, u2, w3, m1, m2, mb)
    return xp[:n]


# speedup vs baseline: 8.3510x; 1.0014x over previous
import jax
import jax.numpy as jnp
from jax import lax
from jax.experimental import pallas as pl
from jax.experimental.pallas import tpu as pltpu
from jax.experimental.pallas import tpu_sc as plsc

_NC = 2
_NS = 16
_RT = _NC * _NS
_L = 16
_CHUNK = 128
_SB = 40
_JR = 256


def _sc_route(nchr, cap_ch, half):
    cap = cap_ch * _CHUNK
    mesh = plsc.VectorSubcoreMesh(core_axis_name="c", subcore_axis_name="s",
                                  num_cores=_NC)

    def body(src_hbm, dst_hbm, lists_hbm, cnt_hbm,
             sidx, didx, l0s, l0d, l0e, l1s, l1d, l1e, cbuf):
        c = lax.axis_index("c")
        s = lax.axis_index("s")
        rt = s * _NC + c
        iota = lax.iota(jnp.int32, _L)

        ept = nchr * _CHUNK

        def superchunk(i, off):
            pltpu.sync_copy(src_hbm.at[rt].at[pl.ds(i * _SB, _SB)], sidx)
            pltpu.sync_copy(dst_hbm.at[rt].at[pl.ds(i * _SB, _SB)], didx)

            def chunk(j, off2):
                o0, o1 = off2
                ebase = rt * ept + (i * _SB + j) * _CHUNK
                for v in range(_CHUNK // _L):
                    sv = sidx[j, pl.ds(v * _L, _L)]
                    dv = didx[j, pl.ds(v * _L, _L)]
                    ev = ebase + v * _L + iota
                    m0 = dv < half
                    m1 = jnp.logical_not(m0)
                    p0 = o0 + plsc.cumsum(m0.astype(jnp.int32)) - 1
                    p1 = o1 + plsc.cumsum(m1.astype(jnp.int32)) - 1
                    plsc.store_scatter(l0s, [p0], sv, mask=m0)
                    plsc.store_scatter(l0d, [p0], dv, mask=m0)
                    plsc.store_scatter(l0e, [p0], ev, mask=m0)
                    plsc.store_scatter(l1s, [p1], sv, mask=m1)
                    plsc.store_scatter(l1d, [p1], dv - half, mask=m1)
                    plsc.store_scatter(l1e, [p1], ev, mask=m1)
                    n0 = jnp.max(plsc.all_reduce_population_count(m0))
                    o0 = o0 + n0
                    o1 = o1 + (_L - n0)
                return (o0, o1)
            return lax.fori_loop(0, _SB, chunk, off)
        o0, o1 = lax.fori_loop(0, nchr // _SB, superchunk,
                               (jnp.int32(0), jnp.int32(0)))

        for v in range(_CHUNK // _L):
            jsrc = (iota + v * _L + rt * 97) % half
            jdst = half + ((iota + v * _L) % _JR)
            jeid = (iota + v * _L + rt * 131) % ept
            l0s[pl.ds(o0 + v * _L, _L)] = jsrc
            l0d[pl.ds(o0 + v * _L, _L)] = jdst
            l0e[pl.ds(o0 + v * _L, _L)] = jeid
            l1s[pl.ds(o1 + v * _L, _L)] = jsrc
            l1d[pl.ds(o1 + v * _L, _L)] = jdst
            l1e[pl.ds(o1 + v * _L, _L)] = jeid
        n0 = (o0 + _CHUNK - 1) // _CHUNK
        n1 = (o1 + _CHUNK - 1) // _CHUNK

        for v in range(_CHUNK // _L):
            cbuf[0, pl.ds(v * _L, _L)] = jnp.full((_L,), n0, jnp.int32)
            cbuf[1, pl.ds(v * _L, _L)] = jnp.full((_L,), n1, jnp.int32)
            for r in range(2, 8):
                cbuf[r, pl.ds(v * _L, _L)] = jnp.zeros((_L,), jnp.int32)
        pltpu.sync_copy(cbuf, cnt_hbm.at[rt])

        base = rt * 6 * cap
        pltpu.sync_copy(l0s, lists_hbm.at[pl.ds(base, cap)])
        pltpu.sync_copy(l0d, lists_hbm.at[pl.ds(base + cap, cap)])
        pltpu.sync_copy(l0e, lists_hbm.at[pl.ds(base + 2 * cap, cap)])
        pltpu.sync_copy(l1s, lists_hbm.at[pl.ds(base + 3 * cap, cap)])
        pltpu.sync_copy(l1d, lists_hbm.at[pl.ds(base + 4 * cap, cap)])
        pltpu.sync_copy(l1e, lists_hbm.at[pl.ds(base + 5 * cap, cap)])

    return pl.kernel(
        body,
        out_type=(jax.ShapeDtypeStruct((_RT * 6 * cap,), jnp.int32),
                  jax.ShapeDtypeStruct((_RT, 8, _CHUNK), jnp.int32)),
        mesh=mesh,
        compiler_params=pltpu.CompilerParams(needs_layout_passes=False),
        scratch_types=[
            pltpu.VMEM((_SB, _CHUNK), jnp.int32),
            pltpu.VMEM((_SB, _CHUNK), jnp.int32),
            pltpu.VMEM((cap,), jnp.int32),
            pltpu.VMEM((cap,), jnp.int32),
            pltpu.VMEM((cap,), jnp.int32),
            pltpu.VMEM((cap,), jnp.int32),
            pltpu.VMEM((cap,), jnp.int32),
            pltpu.VMEM((cap,), jnp.int32),
            pltpu.VMEM((8, _CHUNK), jnp.int32),
        ],
    )


def _sc_gather_routed(n_pad, d, cap_ch, half, gsel):
    cap = cap_ch * _CHUNK
    arows = half + _JR
    rows_per_tile = arows // _NS
    out_rows_per_tile = half // _NS
    mesh = plsc.VectorSubcoreMesh(core_axis_name="c", subcore_axis_name="s",
                                  num_cores=_NC)

    def body(x_hbm, lists_hbm, cnt_hbm, z_hbm, out_hbm,
             ia_s, ia_d, ib_s, ib_d, ic_s, ic_d, id_s, id_d,
             vals0, vals1, cbuf, accum,
             gsem0, gsem1, sema, semb, semc, semd):
        c = lax.axis_index("c")
        s = lax.axis_index("s")
        r0 = s * rows_per_tile

        pltpu.sync_copy(z_hbm, vals0)
        nzfull = rows_per_tile // _CHUNK
        rem = rows_per_tile - nzfull * _CHUNK

        def zrow(i, carry):
            pltpu.sync_copy(vals0, accum.at[pl.ds(r0 + i * _CHUNK, _CHUNK)])
            return carry
        lax.fori_loop(0, nzfull, zrow, 0)
        if rem:
            pltpu.sync_copy(vals0.at[pl.ds(0, rem)],
                            accum.at[pl.ds(r0 + nzfull * _CHUNK, rem)])
        plsc.subcore_barrier()

        rt0 = 2 * s
        rt1 = 2 * s + 1
        pltpu.sync_copy(cnt_hbm.at[rt0], cbuf)
        n0 = jnp.max(jnp.where(c == 0, cbuf[0, pl.ds(0, _L)],
                               cbuf[1, pl.ds(0, _L)]))
        pltpu.sync_copy(cnt_hbm.at[rt1], cbuf)
        n1 = jnp.max(jnp.where(c == 0, cbuf[0, pl.ds(0, _L)],
                               cbuf[1, pl.ds(0, _L)]))
        nt = n0 + n1

        def src_off(i):
            rt = jnp.where(i < n0, rt0, rt1)
            j = jnp.where(i < n0, i, i - n0)
            return (rt * 6 + 3 * c) * cap + j * _CHUNK

        def stage(i, isref, idref, isem):
            off = src_off(i)
            pltpu.async_copy(lists_hbm.at[pl.ds(off + gsel * cap, _CHUNK)],
                             isref, isem)
            pltpu.async_copy(lists_hbm.at[pl.ds(off + cap, _CHUNK)],
                             idref, isem)

        def wait_idx(i, isref, idref, isem):
            off = src_off(i)
            pltpu.make_async_copy(
                lists_hbm.at[pl.ds(off + gsel * cap, _CHUNK)],
                isref, isem).wait()
            pltpu.make_async_copy(lists_hbm.at[pl.ds(off + cap, _CHUNK)],
                                  idref, isem).wait()

        islots = ((ia_s, ia_d, sema), (ib_s, ib_d, semb),
                  (ic_s, ic_d, semc), (id_s, id_d, semd))
        vslots = ((vals0, gsem0), (vals1, gsem1))

        for b in range(4):
            @pl.when(b < nt)
            def _(b=b):
                stage(b, *islots[b])
        hh = _CHUNK // 2

        def gather2(isref, v, gs):
            pltpu.async_copy(x_hbm.at[isref.at[pl.ds(0, hh)]],
                             v.at[pl.ds(0, hh)], gs)
            pltpu.async_copy(x_hbm.at[isref.at[pl.ds(hh, hh)]],
                             v.at[pl.ds(hh, hh)], gs)

        def wait2(isref, v, gs):
            pltpu.make_async_copy(x_hbm.at[isref.at[pl.ds(0, hh)]],
                                  v.at[pl.ds(0, hh)], gs).wait()
            pltpu.make_async_copy(x_hbm.at[isref.at[pl.ds(hh, hh)]],
                                  v.at[pl.ds(hh, hh)], gs).wait()

        for b in range(2):
            @pl.when(b < nt)
            def _(b=b):
                wait_idx(b, *islots[b])
                gather2(islots[b][0], vslots[b][0], vslots[b][1])

        def quad(q, carry):
            j = 4 * q
            for b in range(4):
                jb = j + b
                v, gs = vslots[b % 2]
                isl = islots[b]
                inx = islots[(b + 2) % 4]

                @pl.when(jb < nt)
                def _(jb=jb, v=v, gs=gs, isl=isl, inx=inx):
                    wait2(isl[0], v, gs)
                    pltpu.sync_copy(v, accum.at[isl[1]], add=True)

                    @pl.when(jb + 4 < nt)
                    def _():
                        stage(jb + 4, *isl)

                    @pl.when(jb + 2 < nt)
                    def _():
                        wait_idx(jb + 2, *inx)
                        gather2(inx[0], v, gs)
            return carry
        lax.fori_loop(0, (nt + 3) // 4, quad, 0)
        plsc.subcore_barrier()

        pltpu.sync_copy(
            accum.at[pl.ds(s * out_rows_per_tile, out_rows_per_tile)],
            out_hbm.at[pl.ds(c * half + s * out_rows_per_tile,
                             out_rows_per_tile)])

    return pl.kernel(
        body,
        out_type=jax.ShapeDtypeStruct((_NC * half, d), jnp.float32),
        mesh=mesh,
        compiler_params=pltpu.CompilerParams(needs_layout_passes=False),
        scratch_types=[
            pltpu.VMEM((_CHUNK,), jnp.int32),
            pltpu.VMEM((_CHUNK,), jnp.int32),
            pltpu.VMEM((_CHUNK,), jnp.int32),
            pltpu.VMEM((_CHUNK,), jnp.int32),
            pltpu.VMEM((_CHUNK,), jnp.int32),
            pltpu.VMEM((_CHUNK,), jnp.int32),
            pltpu.VMEM((_CHUNK,), jnp.int32),
            pltpu.VMEM((_CHUNK,), jnp.int32),
            pltpu.VMEM((_CHUNK, d), jnp.float32),
            pltpu.VMEM((_CHUNK, d), jnp.float32),
            pltpu.VMEM((8, _CHUNK), jnp.int32),
            pltpu.VMEM_SHARED((arows, d), jnp.float32),
            pltpu.SemaphoreType.DMA,
            pltpu.SemaphoreType.DMA,
            pltpu.SemaphoreType.DMA,
            pltpu.SemaphoreType.DMA,
            pltpu.SemaphoreType.DMA,
            pltpu.SemaphoreType.DMA,
        ],
    )


def _update_body(x_ref, g_ref, sd_ref, u1_ref, u2_ref, w3_ref,
                 m1_ref, m2_ref, mb_ref, o_ref):
    f32 = jnp.float32
    x = x_ref[...]
    g = g_ref[...]
    sd = sd_ref[...]
    deg = sd[:, 16:17]
    msg = (deg * jnp.dot(x, u1_ref[...], preferred_element_type=f32)
           + jnp.dot(g, u2_ref[...], preferred_element_type=f32)
           + jnp.dot(sd, w3_ref[...], preferred_element_type=f32))
    o_ref[...] = jnp.maximum(
        jnp.dot(x, m1_ref[...], preferred_element_type=f32)
        + jnp.dot(msg, m2_ref[...], preferred_element_type=f32)
        + mb_ref[...], 0.0)


def _tc_update(x, g, sdparts, u1, u2, w3, m1, m2, mb):
    n_pad, d = x.shape
    da = sdparts.shape[1]
    blk = 1024
    grid = n_pad // blk
    full = lambda i: (0, 0)
    row = pl.BlockSpec((blk, d), lambda i: (i, 0))
    return pl.pallas_call(
        _update_body,
        grid=(grid,),
        in_specs=[
            row,
            row,
            pl.BlockSpec((blk, da), lambda i: (i, 0)),
            pl.BlockSpec((d, d), full),
            pl.BlockSpec((d, d), full),
            pl.BlockSpec((da, d), full),
            pl.BlockSpec((d, d), full),
            pl.BlockSpec((d, d), full),
            pl.BlockSpec((1, d), full),
        ],
        out_specs=row,
        out_shape=jax.ShapeDtypeStruct((n_pad, d), jnp.float32),
    )(x, g, sdparts, u1, u2, w3, m1, m2, mb)


def kernel(x, edge_index, edge_attr, U_W, U_b, M_W, M_b):
    n, d = x.shape
    e = edge_index.shape[1]
    de = edge_attr.shape[1]
    t = U_W.shape[0]

    sbe = _CHUNK * _SB
    eptr = -(-e // (_RT * sbe)) * sbe
    nchr = eptr // _CHUNK
    ep = eptr * _RT
    pad = ep - e
    n_pad = -(-(n + 16) // (_CHUNK * _NS)) * (_CHUNK * _NS)
    junk = n_pad - n
    half = n_pad // _NC
    cap_ch = nchr + 1

    src = edge_index[0]
    dst = edge_index[1]
    ar = jnp.arange(pad, dtype=jnp.int32)
    src_p = jnp.concatenate([src, ar % n]).reshape(_RT, nchr, _CHUNK)
    dst_p = jnp.concatenate([dst, n + ar % junk]).reshape(_RT, nchr, _CHUNK)

    da = d
    aug = jnp.concatenate(
        [edge_attr, jnp.ones((e, 1), jnp.float32),
         jnp.zeros((e, da - de - 1), jnp.float32)], axis=1)
    aug_p = jnp.pad(aug, ((0, pad), (0, 0)))

    xp = jnp.pad(x, ((0, junk), (0, 0)))
    zeros = jnp.zeros((_CHUNK, d), jnp.float32)
    zeros_sa = jnp.zeros((_CHUNK, da), jnp.float32)

    lists, cnts = _sc_route(nchr, cap_ch, half)(src_p, dst_p)
    seg_gather = _sc_gather_routed(n_pad, d, cap_ch, half, gsel=0)
    seg_aug = _sc_gather_routed(n_pad, d, cap_ch, half, gsel=2)
    sdparts = seg_aug(aug_p, lists, cnts, zeros_sa)

    for k in range(t):
        u1 = U_W[k, :d]
        u2 = U_W[k, d:2 * d]
        w3 = jnp.zeros((da, d), jnp.float32)
        w3 = w3.at[:de].set(U_W[k, 2 * d:]).at[de].set(U_b[k])
        m1 = M_W[k, :d]
        m2 = M_W[k, d:]
        mb = M_b[k][None, :]
        g = seg_gather(xp, lists, cnts, zeros)
        xp = _tc_update(xp, g, sdparts, u1, u2, w3, m1, m2, mb)
    return xp[:n]


# speedup vs baseline: 8.3640x; 1.0016x over previous
import jax
import jax.numpy as jnp
from jax import lax
from jax.experimental import pallas as pl
from jax.experimental.pallas import tpu as pltpu
from jax.experimental.pallas import tpu_sc as plsc

_NC = 2
_NS = 16
_RT = _NC * _NS
_L = 16
_CHUNK = 128
_SB = 40
_JR = 256


def _sc_route(nchr, cap_ch, half):
    cap = cap_ch * _CHUNK
    mesh = plsc.VectorSubcoreMesh(core_axis_name="c", subcore_axis_name="s",
                                  num_cores=_NC)

    def body(src_hbm, dst_hbm, lists_hbm, cnt_hbm,
             sidx, didx, l0s, l0d, l0e, l1s, l1d, l1e, cbuf):
        c = lax.axis_index("c")
        s = lax.axis_index("s")
        rt = s * _NC + c
        iota = lax.iota(jnp.int32, _L)

        ept = nchr * _CHUNK

        def superchunk(i, off):
            pltpu.sync_copy(src_hbm.at[rt].at[pl.ds(i * _SB, _SB)], sidx)
            pltpu.sync_copy(dst_hbm.at[rt].at[pl.ds(i * _SB, _SB)], didx)

            def chunk(j, off2):
                o0, o1 = off2
                ebase = rt * ept + (i * _SB + j) * _CHUNK
                for v in range(_CHUNK // _L):
                    sv = sidx[j, pl.ds(v * _L, _L)]
                    dv = didx[j, pl.ds(v * _L, _L)]
                    ev = ebase + v * _L + iota
                    m0 = dv < half
                    m1 = jnp.logical_not(m0)
                    p0 = o0 + plsc.cumsum(m0.astype(jnp.int32)) - 1
                    p1 = o1 + plsc.cumsum(m1.astype(jnp.int32)) - 1
                    plsc.store_scatter(l0s, [p0], sv, mask=m0)
                    plsc.store_scatter(l0d, [p0], dv, mask=m0)
                    plsc.store_scatter(l0e, [p0], ev, mask=m0)
                    plsc.store_scatter(l1s, [p1], sv, mask=m1)
                    plsc.store_scatter(l1d, [p1], dv - half, mask=m1)
                    plsc.store_scatter(l1e, [p1], ev, mask=m1)
                    n0 = jnp.max(plsc.all_reduce_population_count(m0))
                    o0 = o0 + n0
                    o1 = o1 + (_L - n0)
                return (o0, o1)
            return lax.fori_loop(0, _SB, chunk, off)
        o0, o1 = lax.fori_loop(0, nchr // _SB, superchunk,
                               (jnp.int32(0), jnp.int32(0)))

        for v in range(_CHUNK // _L):
            jsrc = (iota + v * _L + rt * 97) % half
            jdst = half + ((iota + v * _L) % _JR)
            jeid = (iota + v * _L + rt * 131) % ept
            l0s[pl.ds(o0 + v * _L, _L)] = jsrc
            l0d[pl.ds(o0 + v * _L, _L)] = jdst
            l0e[pl.ds(o0 + v * _L, _L)] = jeid
            l1s[pl.ds(o1 + v * _L, _L)] = jsrc
            l1d[pl.ds(o1 + v * _L, _L)] = jdst
            l1e[pl.ds(o1 + v * _L, _L)] = jeid
        n0 = (o0 + _CHUNK - 1) // _CHUNK
        n1 = (o1 + _CHUNK - 1) // _CHUNK

        for v in range(_CHUNK // _L):
            cbuf[0, pl.ds(v * _L, _L)] = jnp.full((_L,), n0, jnp.int32)
            cbuf[1, pl.ds(v * _L, _L)] = jnp.full((_L,), n1, jnp.int32)
            for r in range(2, 8):
                cbuf[r, pl.ds(v * _L, _L)] = jnp.zeros((_L,), jnp.int32)
        pltpu.sync_copy(cbuf, cnt_hbm.at[rt])

        base = rt * 6 * cap
        pltpu.sync_copy(l0s, lists_hbm.at[pl.ds(base, cap)])
        pltpu.sync_copy(l0d, lists_hbm.at[pl.ds(base + cap, cap)])
        pltpu.sync_copy(l0e, lists_hbm.at[pl.ds(base + 2 * cap, cap)])
        pltpu.sync_copy(l1s, lists_hbm.at[pl.ds(base + 3 * cap, cap)])
        pltpu.sync_copy(l1d, lists_hbm.at[pl.ds(base + 4 * cap, cap)])
        pltpu.sync_copy(l1e, lists_hbm.at[pl.ds(base + 5 * cap, cap)])

    return pl.kernel(
        body,
        out_type=(jax.ShapeDtypeStruct((_RT * 6 * cap,), jnp.int32),
                  jax.ShapeDtypeStruct((_RT, 8, _CHUNK), jnp.int32)),
        mesh=mesh,
        compiler_params=pltpu.CompilerParams(needs_layout_passes=False, skip_device_barrier=True),
        scratch_types=[
            pltpu.VMEM((_SB, _CHUNK), jnp.int32),
            pltpu.VMEM((_SB, _CHUNK), jnp.int32),
            pltpu.VMEM((cap,), jnp.int32),
            pltpu.VMEM((cap,), jnp.int32),
            pltpu.VMEM((cap,), jnp.int32),
            pltpu.VMEM((cap,), jnp.int32),
            pltpu.VMEM((cap,), jnp.int32),
            pltpu.VMEM((cap,), jnp.int32),
            pltpu.VMEM((8, _CHUNK), jnp.int32),
        ],
    )


def _sc_gather_routed(n_pad, d, cap_ch, half, gsel):
    cap = cap_ch * _CHUNK
    arows = half + _JR
    rows_per_tile = arows // _NS
    out_rows_per_tile = half // _NS
    mesh = plsc.VectorSubcoreMesh(core_axis_name="c", subcore_axis_name="s",
                                  num_cores=_NC)

    def body(x_hbm, lists_hbm, cnt_hbm, z_hbm, out_hbm,
             ia_s, ia_d, ib_s, ib_d, ic_s, ic_d, id_s, id_d,
             vals0, vals1, cbuf, accum,
             gsem0, gsem1, sema, semb, semc, semd):
        c = lax.axis_index("c")
        s = lax.axis_index("s")
        r0 = s * rows_per_tile

        pltpu.sync_copy(z_hbm, vals0)
        nzfull = rows_per_tile // _CHUNK
        rem = rows_per_tile - nzfull * _CHUNK

        def zrow(i, carry):
            pltpu.sync_copy(vals0, accum.at[pl.ds(r0 + i * _CHUNK, _CHUNK)])
            return carry
        lax.fori_loop(0, nzfull, zrow, 0)
        if rem:
            pltpu.sync_copy(vals0.at[pl.ds(0, rem)],
                            accum.at[pl.ds(r0 + nzfull * _CHUNK, rem)])
        plsc.subcore_barrier()

        rt0 = 2 * s
        rt1 = 2 * s + 1
        pltpu.sync_copy(cnt_hbm.at[rt0], cbuf)
        n0 = jnp.max(jnp.where(c == 0, cbuf[0, pl.ds(0, _L)],
                               cbuf[1, pl.ds(0, _L)]))
        pltpu.sync_copy(cnt_hbm.at[rt1], cbuf)
        n1 = jnp.max(jnp.where(c == 0, cbuf[0, pl.ds(0, _L)],
                               cbuf[1, pl.ds(0, _L)]))
        nt = n0 + n1

        def src_off(i):
            rt = jnp.where(i < n0, rt0, rt1)
            j = jnp.where(i < n0, i, i - n0)
            return (rt * 6 + 3 * c) * cap + j * _CHUNK

        def stage(i, isref, idref, isem):
            off = src_off(i)
            pltpu.async_copy(lists_hbm.at[pl.ds(off + gsel * cap, _CHUNK)],
                             isref, isem)
            pltpu.async_copy(lists_hbm.at[pl.ds(off + cap, _CHUNK)],
                             idref, isem)

        def wait_idx(i, isref, idref, isem):
            off = src_off(i)
            pltpu.make_async_copy(
                lists_hbm.at[pl.ds(off + gsel * cap, _CHUNK)],
                isref, isem).wait()
            pltpu.make_async_copy(lists_hbm.at[pl.ds(off + cap, _CHUNK)],
                                  idref, isem).wait()

        islots = ((ia_s, ia_d, sema), (ib_s, ib_d, semb),
                  (ic_s, ic_d, semc), (id_s, id_d, semd))
        vslots = ((vals0, gsem0), (vals1, gsem1))

        for b in range(4):
            @pl.when(b < nt)
            def _(b=b):
                stage(b, *islots[b])
        hh = _CHUNK // 2

        def gather2(isref, v, gs):
            pltpu.async_copy(x_hbm.at[isref.at[pl.ds(0, hh)]],
                             v.at[pl.ds(0, hh)], gs)
            pltpu.async_copy(x_hbm.at[isref.at[pl.ds(hh, hh)]],
                             v.at[pl.ds(hh, hh)], gs)

        def wait2(isref, v, gs):
            pltpu.make_async_copy(x_hbm.at[isref.at[pl.ds(0, hh)]],
                                  v.at[pl.ds(0, hh)], gs).wait()
            pltpu.make_async_copy(x_hbm.at[isref.at[pl.ds(hh, hh)]],
                                  v.at[pl.ds(hh, hh)], gs).wait()

        for b in range(2):
            @pl.when(b < nt)
            def _(b=b):
                wait_idx(b, *islots[b])
                gather2(islots[b][0], vslots[b][0], vslots[b][1])

        def quad(q, carry):
            j = 4 * q
            for b in range(4):
                jb = j + b
                v, gs = vslots[b % 2]
                isl = islots[b]
                inx = islots[(b + 2) % 4]

                @pl.when(jb < nt)
                def _(jb=jb, v=v, gs=gs, isl=isl, inx=inx):
                    wait2(isl[0], v, gs)
                    pltpu.sync_copy(v, accum.at[isl[1]], add=True)

                    @pl.when(jb + 4 < nt)
                    def _():
                        stage(jb + 4, *isl)

                    @pl.when(jb + 2 < nt)
                    def _():
                        wait_idx(jb + 2, *inx)
                        gather2(inx[0], v, gs)
            return carry
        lax.fori_loop(0, (nt + 3) // 4, quad, 0)
        plsc.subcore_barrier()

        pltpu.sync_copy(
            accum.at[pl.ds(s * out_rows_per_tile, out_rows_per_tile)],
            out_hbm.at[pl.ds(c * half + s * out_rows_per_tile,
                             out_rows_per_tile)])

    return pl.kernel(
        body,
        out_type=jax.ShapeDtypeStruct((_NC * half, d), jnp.float32),
        mesh=mesh,
        compiler_params=pltpu.CompilerParams(needs_layout_passes=False, skip_device_barrier=True),
        scratch_types=[
            pltpu.VMEM((_CHUNK,), jnp.int32),
            pltpu.VMEM((_CHUNK,), jnp.int32),
            pltpu.VMEM((_CHUNK,), jnp.int32),
            pltpu.VMEM((_CHUNK,), jnp.int32),
            pltpu.VMEM((_CHUNK,), jnp.int32),
            pltpu.VMEM((_CHUNK,), jnp.int32),
            pltpu.VMEM((_CHUNK,), jnp.int32),
            pltpu.VMEM((_CHUNK,), jnp.int32),
            pltpu.VMEM((_CHUNK, d), jnp.float32),
            pltpu.VMEM((_CHUNK, d), jnp.float32),
            pltpu.VMEM((8, _CHUNK), jnp.int32),
            pltpu.VMEM_SHARED((arows, d), jnp.float32),
            pltpu.SemaphoreType.DMA,
            pltpu.SemaphoreType.DMA,
            pltpu.SemaphoreType.DMA,
            pltpu.SemaphoreType.DMA,
            pltpu.SemaphoreType.DMA,
            pltpu.SemaphoreType.DMA,
        ],
    )


def _update_body(x_ref, g_ref, sd_ref, u1_ref, u2_ref, w3_ref,
                 m1_ref, m2_ref, mb_ref, o_ref):
    f32 = jnp.float32
    x = x_ref[...]
    g = g_ref[...]
    sd = sd_ref[...]
    deg = sd[:, 16:17]
    msg = (deg * jnp.dot(x, u1_ref[...], preferred_element_type=f32)
           + jnp.dot(g, u2_ref[...], preferred_element_type=f32)
           + jnp.dot(sd, w3_ref[...], preferred_element_type=f32))
    o_ref[...] = jnp.maximum(
        jnp.dot(x, m1_ref[...], preferred_element_type=f32)
        + jnp.dot(msg, m2_ref[...], preferred_element_type=f32)
        + mb_ref[...], 0.0)


def _tc_update(x, g, sdparts, u1, u2, w3, m1, m2, mb):
    n_pad, d = x.shape
    da = sdparts.shape[1]
    blk = 1024
    grid = n_pad // blk
    full = lambda i: (0, 0)
    row = pl.BlockSpec((blk, d), lambda i: (i, 0))
    return pl.pallas_call(
        _update_body,
        grid=(grid,),
        in_specs=[
            row,
            row,
            pl.BlockSpec((blk, da), lambda i: (i, 0)),
            pl.BlockSpec((d, d), full),
            pl.BlockSpec((d, d), full),
            pl.BlockSpec((da, d), full),
            pl.BlockSpec((d, d), full),
            pl.BlockSpec((d, d), full),
            pl.BlockSpec((1, d), full),
        ],
        out_specs=row,
        out_shape=jax.ShapeDtypeStruct((n_pad, d), jnp.float32),
    )(x, g, sdparts, u1, u2, w3, m1, m2, mb)


def kernel(x, edge_index, edge_attr, U_W, U_b, M_W, M_b):
    n, d = x.shape
    e = edge_index.shape[1]
    de = edge_attr.shape[1]
    t = U_W.shape[0]

    sbe = _CHUNK * _SB
    eptr = -(-e // (_RT * sbe)) * sbe
    nchr = eptr // _CHUNK
    ep = eptr * _RT
    pad = ep - e
    n_pad = -(-(n + 16) // (_CHUNK * _NS)) * (_CHUNK * _NS)
    junk = n_pad - n
    half = n_pad // _NC
    cap_ch = nchr + 1

    src = edge_index[0]
    dst = edge_index[1]
    ar = jnp.arange(pad, dtype=jnp.int32)
    src_p = jnp.concatenate([src, ar % n]).reshape(_RT, nchr, _CHUNK)
    dst_p = jnp.concatenate([dst, n + ar % junk]).reshape(_RT, nchr, _CHUNK)

    da = d
    aug = jnp.concatenate(
        [edge_attr, jnp.ones((e, 1), jnp.float32),
         jnp.zeros((e, da - de - 1), jnp.float32)], axis=1)
    aug_p = jnp.pad(aug, ((0, pad), (0, 0)))

    xp = jnp.pad(x, ((0, junk), (0, 0)))
    zeros = jnp.zeros((_CHUNK, d), jnp.float32)
    zeros_sa = jnp.zeros((_CHUNK, da), jnp.float32)

    lists, cnts = _sc_route(nchr, cap_ch, half)(src_p, dst_p)
    seg_gather = _sc_gather_routed(n_pad, d, cap_ch, half, gsel=0)
    seg_aug = _sc_gather_routed(n_pad, d, cap_ch, half, gsel=2)
    sdparts = seg_aug(aug_p, lists, cnts, zeros_sa)

    for k in range(t):
        u1 = U_W[k, :d]
        u2 = U_W[k, d:2 * d]
        w3 = jnp.zeros((da, d), jnp.float32)
        w3 = w3.at[:de].set(U_W[k, 2 * d:]).at[de].set(U_b[k])
        m1 = M_W[k, :d]
        m2 = M_W[k, d:]
        mb = M_b[k][None, :]
        g = seg_gather(xp, lists, cnts, zeros)
        xp = _tc_update(xp, g, sdparts, u1, u2, w3, m1, m2, mb)
    return xp[:n]
